# baseline (speedup 1.0000x reference)
import functools

import jax
import jax.numpy as jnp
import numpy as np
from jax import lax
from jax.experimental import pallas as pl
from jax.experimental.pallas import tpu as pltpu
from jax.experimental.pallas import tpu_sc as plsc

N = 10000
E = 320000
D = 128
D2 = D // 2
R = 4

NC = 2
NS = 16
L = 16
NW = NC * NS
CH = 80
NCHUNK = E // CH
NJ = -(-NCHUNK // NW)
NJP = 2 * (-(-NJ // 2))
BLK = 80
NBLK = N // BLK
NT = -(-NBLK // NS)

_PERM = np.empty(D, np.int32)
for _t in range(D // 32):
    for _i in range(16):
        _PERM[32 * _t + _i] = 32 * _t + 2 * _i
        _PERM[32 * _t + 16 + _i] = 32 * _t + 2 * _i + 1


def _relmm(x, rel_weight):
    BN = 1000

    def body(x_ref, w_ref, o_ref):
        xb = x_ref[...]
        for r in range(R):
            o_ref[r] = jnp.dot(
                xb, w_ref[r], preferred_element_type=jnp.float32
            ).astype(jnp.bfloat16)

    return pl.pallas_call(
        body,
        grid=(N // BN,),
        in_specs=[
            pl.BlockSpec((BN, D), lambda i: (i, 0)),
            pl.BlockSpec((R, D, D), lambda i: (0, 0, 0)),
        ],
        out_specs=pl.BlockSpec((R, BN, D), lambda i: (0, i, 0)),
        out_shape=jax.ShapeDtypeStruct((R, N, D), jnp.bfloat16),
    )(x, rel_weight)


def _sc_agg(xw32, ei, et):
    mesh = plsc.VectorSubcoreMesh(core_axis_name="c", subcore_axis_name="s")

    @functools.partial(
        pl.kernel,
        mesh=mesh,
        compiler_params=pltpu.CompilerParams(needs_layout_passes=False,
                                             use_tc_tiling_on_sc=False),
        out_type=jax.ShapeDtypeStruct((NC, N, D), jnp.float32),
        scratch_types=[
            pltpu.VMEM((CH,), jnp.int32),
            pltpu.VMEM((CH,), jnp.int32),
            pltpu.VMEM((CH,), jnp.int32),
            pltpu.VMEM((CH,), jnp.int32),
            pltpu.VMEM((CH,), jnp.int32),
            pltpu.VMEM((CH,), jnp.int32),
            pltpu.VMEM((CH,), jnp.int32),
            pltpu.VMEM((CH,), jnp.int32),
            pltpu.VMEM((CH,), jnp.int32),
            pltpu.VMEM((CH,), jnp.int32),
            pltpu.VMEM((CH, D2), jnp.int32),
            pltpu.VMEM((CH, D2), jnp.int32),
            pltpu.VMEM((CH, D), jnp.float32),
            pltpu.VMEM((CH, D), jnp.float32),
            pltpu.VMEM_SHARED((N, D), jnp.float32),
            pltpu.SemaphoreType.DMA,
            pltpu.SemaphoreType.DMA,
            pltpu.SemaphoreType.DMA,
        ],
    )
    def k(xw_hbm, ei_hbm, et_hbm, out_hbm,
          srcA, srcB, etA, etB, drwA, drwB, gixA, gixB, dstA, dstB,
          rpA, rpB, rfA, rfB, agg_sh, isem, gsem, ssem):
        c = lax.axis_index("c")
        s = lax.axis_index("s")
        wid = s * NC + c

        z16 = jnp.zeros((L,), jnp.float32)

        def zrow(i, carry):
            for j in range(D // L):
                rfA[i, pl.ds(j * L, L)] = z16
            return carry

        lax.fori_loop(0, BLK, zrow, 0)
        for t in range(NT):
            b = t * NS + s

            @pl.when(b < NBLK)
            def _():
                pltpu.sync_copy(rfA, agg_sh.at[pl.ds(b * BLK, BLK)])

        plsc.subcore_barrier()

        def build(src_v, et_v, drw_v, gix_v, dst_v):
            for i in range(CH // L):
                sl = pl.ds(i * L, L)
                gix_v[sl] = et_v[sl] * N + src_v[sl]
                dst_v[sl] = drw_v[sl]

        def fetch_idx(cid, src_v, et_v, drw_v):
            base = cid * CH
            pltpu.async_copy(ei_hbm.at[pl.ds(base, CH)], src_v, isem)
            pltpu.async_copy(et_hbm.at[pl.ds(base, CH)], et_v, isem)
            pltpu.async_copy(ei_hbm.at[pl.ds(E + base, CH)], drw_v, isem)

        def drain_idx(src_v, et_v, drw_v):
            pltpu.make_async_copy(ei_hbm.at[pl.ds(0, CH)], src_v, isem).wait()
            pltpu.make_async_copy(et_hbm.at[pl.ds(0, CH)], et_v, isem).wait()
            pltpu.make_async_copy(ei_hbm.at[pl.ds(0, CH)], drw_v, isem).wait()

        def unpack_rows(rp_v, rf_v):
            def row(r, carry):
                for t in range(D2 // L):
                    wv = rp_v[r, pl.ds(t * L, L)]
                    bf = plsc.bitcast(wv, jnp.bfloat16)
                    a, b = plsc.unpack(bf, format=plsc.PackFormat.INTERLEAVED)
                    rf_v[r, pl.ds(2 * t * L, L)] = a
                    rf_v[r, pl.ds((2 * t + 1) * L, L)] = b
                return carry

            lax.fori_loop(0, CH, row, 0)

        fetch_idx(wid, srcA, etA, drwA)
        drain_idx(srcA, etA, drwA)
        build(srcA, etA, drwA, gixA, dstA)
        pltpu.async_copy(xw_hbm.at[gixA], rpA, gsem)
        fetch_idx(wid + NW, srcB, etB, drwB)

        def half(j, cur):
            src_c, et_c, drw_c, gix_c, dst_c, rp_c, rf_c = (
                (srcA, etA, drwA, gixA, dstA, rpA, rfA) if cur == 0
                else (srcB, etB, drwB, gixB, dstB, rpB, rfB))
            src_n, et_n, drw_n, gix_n, dst_n, rp_n, rf_n = (
                (srcB, etB, drwB, gixB, dstB, rpB, rfB) if cur == 0
                else (srcA, etA, drwA, gixA, dstA, rpA, rfA))
            cid0 = wid + j * NW
            cid1 = wid + (j + 1) * NW
            cid2 = wid + (j + 2) * NW

            @pl.when((j >= 1) & (wid + (j - 1) * NW < NCHUNK))
            def _():
                pltpu.make_async_copy(rf_n, agg_sh.at[dst_n], ssem).wait()

            @pl.when(cid1 < NCHUNK)
            def _():
                drain_idx(src_n, et_n, drw_n)
                build(src_n, et_n, drw_n, gix_n, dst_n)
                pltpu.async_copy(xw_hbm.at[gix_n], rp_n, gsem)

            @pl.when(cid0 < NCHUNK)
            def _():
                pltpu.make_async_copy(xw_hbm.at[gix_c], rp_c, gsem).wait()
                unpack_rows(rp_c, rf_c)
                pltpu.async_copy(rf_c, agg_sh.at[dst_c], ssem, add=True)

            @pl.when(cid2 < NCHUNK)
            def _():
                fetch_idx(cid2, src_c, et_c, drw_c)

        def body(jj, carry):
            half(2 * jj, 0)
            half(2 * jj + 1, 1)
            return carry

        lax.fori_loop(0, NJP // 2, body, 0)
        plsc.subcore_barrier()

        for t in range(NT):
            b = t * NS + s

            @pl.when(b < NBLK)
            def _():
                r0 = b * BLK
                pltpu.sync_copy(agg_sh.at[pl.ds(r0, BLK)],
                                out_hbm.at[c, pl.ds(r0, BLK)])

    return k(xw32, ei, et)


def _mlp(x, partials, lw_s, rb_s, w1xt, w1mt_s, b1, w2xt, w2mt, b2):
    BN = 1000

    def body(x_ref, p_ref, lw_ref, rb_ref, w1x_ref, w1m_ref, b1_ref,
             w2x_ref, w2m_ref, b2_ref, o_ref):
        xb = x_ref[...]
        msg = (p_ref[0] + p_ref[1]
               + jnp.dot(xb, lw_ref[...], preferred_element_type=jnp.float32)
               + rb_ref[...])
        h = (jnp.dot(xb, w1x_ref[...], preferred_element_type=jnp.float32)
             + jnp.dot(msg, w1m_ref[...], preferred_element_type=jnp.float32)
             + b1_ref[...])
        mid = jnp.tanh(h)
        o_ref[...] = (jnp.dot(xb, w2x_ref[...], preferred_element_type=jnp.float32)
                      + jnp.dot(mid, w2m_ref[...], preferred_element_type=jnp.float32)
                      + b2_ref[...])

    return pl.pallas_call(
        body,
        grid=(N // BN,),
        in_specs=[
            pl.BlockSpec((BN, D), lambda i: (i, 0)),
            pl.BlockSpec((NC, BN, D), lambda i: (0, i, 0)),
            pl.BlockSpec((D, D), lambda i: (0, 0)),
            pl.BlockSpec((1, D), lambda i: (0, 0)),
            pl.BlockSpec((D, 2 * D), lambda i: (0, 0)),
            pl.BlockSpec((D, 2 * D), lambda i: (0, 0)),
            pl.BlockSpec((1, 2 * D), lambda i: (0, 0)),
            pl.BlockSpec((D, D), lambda i: (0, 0)),
            pl.BlockSpec((2 * D, D), lambda i: (0, 0)),
            pl.BlockSpec((1, D), lambda i: (0, 0)),
        ],
        out_specs=pl.BlockSpec((BN, D), lambda i: (i, 0)),
        out_shape=jax.ShapeDtypeStruct((N, D), jnp.float32),
    )(x, partials, lw_s, rb_s.reshape(1, D), w1xt, w1mt_s,
      b1.reshape(1, 2 * D), w2xt, w2mt, b2.reshape(1, D))


def kernel(x, edge_index, edges_type, is_block, rel_weight, loop_weight,
           rel_bias, W1, b1, W2, b2):
    del is_block
    ei = edge_index.astype(jnp.int32).reshape(2 * E)
    et = edges_type.astype(jnp.int32)
    xw = _relmm(x, rel_weight)
    xw32 = jax.lax.bitcast_convert_type(
        xw.reshape(R * N, D2, 2), jnp.int32)
    partials = _sc_agg(xw32, ei, et)
    perm = jnp.asarray(_PERM)
    lw_s = loop_weight[:, perm]
    rb_s = rel_bias[perm]
    w1xt = W1[:, :D].T
    w1mt_s = W1[:, D:].T[perm]
    w2xt = W2[:, :D].T
    w2mt = W2[:, D:].T
    return _mlp(x, partials, lw_s, rb_s, w1xt, w1mt_s, b1,
                w2xt, w2mt, b2)


# speedup vs baseline: 2.8984x; 2.8984x over previous
import functools

import jax
import jax.numpy as jnp
from jax import lax
from jax.experimental import pallas as pl
from jax.experimental.pallas import tpu as pltpu
from jax.experimental.pallas import tpu_sc as plsc

N = 10000
E = 320000
D = 128
R = 4

NC = 2
NS = 16
L = 16
NW = NC * NS
CH = 128
K = 1
CHOUT = K * CH
NCHUNK = E // CHOUT
NJ = -(-NCHUNK // NW)
NJP = 2 * (-(-NJ // 2))
IW = 3 * CHOUT
BLK = 80
NBLK = N // BLK
NT = -(-NBLK // NS)


def _relmm(x, rel_weight):
    BN = 1000

    def body(x_ref, w_ref, o_ref):
        xb = x_ref[...]
        for r in range(R):
            o_ref[r] = jnp.dot(xb, w_ref[r],
                               preferred_element_type=jnp.float32)

    return pl.pallas_call(
        body,
        grid=(N // BN,),
        in_specs=[
            pl.BlockSpec((BN, D), lambda i: (i, 0)),
            pl.BlockSpec((R, D, D), lambda i: (0, 0, 0)),
        ],
        out_specs=pl.BlockSpec((R, BN, D), lambda i: (0, i, 0)),
        out_shape=jax.ShapeDtypeStruct((R, N, D), jnp.float32),
    )(x, rel_weight)


def _sc_agg(xw, ei, et):
    mesh = plsc.VectorSubcoreMesh(core_axis_name="c", subcore_axis_name="s")

    @functools.partial(
        pl.kernel,
        mesh=mesh,
        out_type=jax.ShapeDtypeStruct((NC, N, D), jnp.float32),
        scratch_types=[
            pltpu.VMEM((CH,), jnp.int32),
            pltpu.VMEM((CH,), jnp.int32),
            pltpu.VMEM((CH,), jnp.int32),
            pltpu.VMEM((CH,), jnp.int32),
            pltpu.VMEM((CH,), jnp.int32),
            pltpu.VMEM((CH,), jnp.int32),
            pltpu.VMEM((CH,), jnp.int32),
            pltpu.VMEM((CH,), jnp.int32),
            pltpu.VMEM((CH,), jnp.int32),
            pltpu.VMEM((CH,), jnp.int32),
            pltpu.VMEM((CHOUT, D), jnp.float32),
            pltpu.VMEM((CHOUT, D), jnp.float32),
            pltpu.VMEM_SHARED((N, D), jnp.float32),
            pltpu.SemaphoreType.DMA,
            pltpu.SemaphoreType.DMA,
            pltpu.SemaphoreType.DMA,
        ],
    )
    def k(xw_hbm, ei_hbm, et_hbm, out_hbm,
          srcA, srcB, etA, etB, drwA, drwB, gixA, gixB, dstA, dstB,
          rowA, rowB, agg_sh, isem, gsem, ssem):
        c = lax.axis_index("c")
        s = lax.axis_index("s")
        wid = s * NC + c
        stage_v = rowA.at[pl.ds(0, BLK)]

        z16 = jnp.zeros((L,), jnp.float32)

        def zrow(i, carry):
            for j in range(D // L):
                rowA[i, pl.ds(j * L, L)] = z16
            return carry

        lax.fori_loop(0, BLK, zrow, 0)
        for t in range(NT):
            b = t * NS + s

            @pl.when(b < NBLK)
            def _():
                pltpu.sync_copy(stage_v, agg_sh.at[pl.ds(b * BLK, BLK)])

        plsc.subcore_barrier()

        def build(src_v, et_v, drw_v, gix_v, dst_v):
            for i in range(CH // L):
                sl = pl.ds(i * L, L)
                gix_v[sl] = et_v[sl] * N + src_v[sl]
                dst_v[sl] = drw_v[sl]

        def fetch_idx(cid, src_v, et_v, drw_v):
            base = cid * CH
            pltpu.async_copy(ei_hbm.at[pl.ds(base, CH)], src_v, isem)
            pltpu.async_copy(et_hbm.at[pl.ds(base, CH)], et_v, isem)
            pltpu.async_copy(ei_hbm.at[pl.ds(E + base, CH)], drw_v, isem)

        def drain_idx(src_v, et_v, drw_v):
            pltpu.make_async_copy(ei_hbm.at[pl.ds(0, CH)], src_v, isem).wait()
            pltpu.make_async_copy(et_hbm.at[pl.ds(0, CH)], et_v, isem).wait()
            pltpu.make_async_copy(ei_hbm.at[pl.ds(0, CH)], drw_v, isem).wait()

        fetch_idx(wid, srcA, etA, drwA)
        drain_idx(srcA, etA, drwA)
        build(srcA, etA, drwA, gixA, dstA)
        pltpu.async_copy(xw_hbm.at[gixA], rowA, gsem)
        fetch_idx(wid + NW, srcB, etB, drwB)

        def half(j, cur):
            src_c, et_c, drw_c, gix_c, dst_c, row_c = (
                (srcA, etA, drwA, gixA, dstA, rowA) if cur == 0
                else (srcB, etB, drwB, gixB, dstB, rowB))
            src_n, et_n, drw_n, gix_n, dst_n, row_n = (
                (srcB, etB, drwB, gixB, dstB, rowB) if cur == 0
                else (srcA, etA, drwA, gixA, dstA, rowA))
            cid0 = wid + j * NW
            cid1 = wid + (j + 1) * NW
            cid2 = wid + (j + 2) * NW

            @pl.when((j >= 1) & (wid + (j - 1) * NW < NCHUNK))
            def _():
                pltpu.make_async_copy(row_n, agg_sh.at[dst_n], ssem).wait()

            @pl.when(cid1 < NCHUNK)
            def _():
                drain_idx(src_n, et_n, drw_n)
                build(src_n, et_n, drw_n, gix_n, dst_n)

            @pl.when(cid0 < NCHUNK)
            def _():
                pltpu.make_async_copy(xw_hbm.at[gix_c], row_c, gsem).wait()
                pltpu.async_copy(row_c, agg_sh.at[dst_c], ssem, add=True)

            @pl.when(cid1 < NCHUNK)
            def _():
                pltpu.async_copy(xw_hbm.at[gix_n], row_n, gsem)

            @pl.when(cid2 < NCHUNK)
            def _():
                fetch_idx(cid2, src_c, et_c, drw_c)

        def body(jj, carry):
            half(2 * jj, 0)
            half(2 * jj + 1, 1)
            return carry

        lax.fori_loop(0, NJP // 2, body, 0)
        plsc.subcore_barrier()

        for t in range(NT):
            b = t * NS + s

            @pl.when(b < NBLK)
            def _():
                r0 = b * BLK
                pltpu.sync_copy(agg_sh.at[pl.ds(r0, BLK)],
                                out_hbm.at[c, pl.ds(r0, BLK)])

    return k(xw, ei, et)


def _mlp(x, partials, lw, rb, w1xt, w1mt, b1, w2xt, w2mt, b2):
    BN = 1000

    def body(x_ref, p_ref, lw_ref, rb_ref, w1x_ref, w1m_ref, b1_ref,
             w2x_ref, w2m_ref, b2_ref, o_ref):
        xb = x_ref[...]
        msg = (p_ref[0] + p_ref[1]
               + jnp.dot(xb, lw_ref[...], preferred_element_type=jnp.float32)
               + rb_ref[...])
        h = (jnp.dot(xb, w1x_ref[...], preferred_element_type=jnp.float32)
             + jnp.dot(msg, w1m_ref[...], preferred_element_type=jnp.float32)
             + b1_ref[...])
        mid = jnp.tanh(h)
        o_ref[...] = (jnp.dot(xb, w2x_ref[...], preferred_element_type=jnp.float32)
                      + jnp.dot(mid, w2m_ref[...], preferred_element_type=jnp.float32)
                      + b2_ref[...])

    return pl.pallas_call(
        body,
        grid=(N // BN,),
        in_specs=[
            pl.BlockSpec((BN, D), lambda i: (i, 0)),
            pl.BlockSpec((NC, BN, D), lambda i: (0, i, 0)),
            pl.BlockSpec((D, D), lambda i: (0, 0)),
            pl.BlockSpec((1, D), lambda i: (0, 0)),
            pl.BlockSpec((D, 2 * D), lambda i: (0, 0)),
            pl.BlockSpec((D, 2 * D), lambda i: (0, 0)),
            pl.BlockSpec((1, 2 * D), lambda i: (0, 0)),
            pl.BlockSpec((D, D), lambda i: (0, 0)),
            pl.BlockSpec((2 * D, D), lambda i: (0, 0)),
            pl.BlockSpec((1, D), lambda i: (0, 0)),
        ],
        out_specs=pl.BlockSpec((BN, D), lambda i: (i, 0)),
        out_shape=jax.ShapeDtypeStruct((N, D), jnp.float32),
    )(x, partials, lw, rb.reshape(1, D), w1xt, w1mt, b1.reshape(1, 2 * D),
      w2xt, w2mt, b2.reshape(1, D))


def kernel(x, edge_index, edges_type, is_block, rel_weight, loop_weight,
           rel_bias, W1, b1, W2, b2):
    del is_block
    ei = edge_index.astype(jnp.int32).reshape(2 * E)
    et = edges_type.astype(jnp.int32)
    xw = _relmm(x, rel_weight).reshape(R * N, D)
    partials = _sc_agg(xw, ei, et)
    w1xt = W1[:, :D].T
    w1mt = W1[:, D:].T
    w2xt = W2[:, :D].T
    w2mt = W2[:, D:].T
    return _mlp(x, partials, loop_weight, rel_bias, w1xt, w1mt, b1,
                w2xt, w2mt, b2)


# speedup vs baseline: 2.8995x; 1.0004x over previous
import functools

import jax
import jax.numpy as jnp
from jax import lax
from jax.experimental import pallas as pl
from jax.experimental.pallas import tpu as pltpu
from jax.experimental.pallas import tpu_sc as plsc

N = 10000
E = 320000
D = 128
R = 4

NC = 2
NS = 16
L = 16
NW = NC * NS
CH = 128
NCHUNK = E // CH
NJ = -(-NCHUNK // NW)
NJP = 2 * (-(-NJ // 2))
BLK = 80
NBLK = N // BLK
NT = -(-NBLK // NS)


def _relmm(x, rel_weight):
    BN = 1000

    def body(x_ref, w_ref, o_ref):
        xb = x_ref[...]
        for r in range(R):
            o_ref[r] = jnp.dot(xb, w_ref[r],
                               preferred_element_type=jnp.float32)

    return pl.pallas_call(
        body,
        grid=(N // BN,),
        in_specs=[
            pl.BlockSpec((BN, D), lambda i: (i, 0)),
            pl.BlockSpec((R, D, D), lambda i: (0, 0, 0)),
        ],
        out_specs=pl.BlockSpec((R, BN, D), lambda i: (0, i, 0)),
        out_shape=jax.ShapeDtypeStruct((R, N, D), jnp.float32),
    )(x, rel_weight)


def _sc_agg(xw, ei, et):
    mesh = plsc.VectorSubcoreMesh(core_axis_name="c", subcore_axis_name="s")

    @functools.partial(
        pl.kernel,
        mesh=mesh,
        out_type=jax.ShapeDtypeStruct((NC, N, D), jnp.float32),
        scratch_types=[
            pltpu.VMEM((CH,), jnp.int32),
            pltpu.VMEM((CH,), jnp.int32),
            pltpu.VMEM((CH,), jnp.int32),
            pltpu.VMEM((CH,), jnp.int32),
            pltpu.VMEM((CH,), jnp.int32),
            pltpu.VMEM((CH,), jnp.int32),
            pltpu.VMEM((CH,), jnp.int32),
            pltpu.VMEM((CH,), jnp.int32),
            pltpu.VMEM((CH,), jnp.int32),
            pltpu.VMEM((CH,), jnp.int32),
            pltpu.VMEM((CH, D), jnp.float32),
            pltpu.VMEM((CH, D), jnp.float32),
            pltpu.VMEM_SHARED((N, D), jnp.float32),
            pltpu.SemaphoreType.DMA,
            pltpu.SemaphoreType.DMA,
            pltpu.SemaphoreType.DMA,
        ],
    )
    def k(xw_hbm, ei_hbm, et_hbm, out_hbm,
          srcA, srcB, etA, etB, drwA, drwB, gixA, gixB, dstA, dstB,
          rowA, rowB, agg_sh, isem, gsem, ssem):
        c = lax.axis_index("c")
        s = lax.axis_index("s")
        wid = s * NC + c
        stage_v = rowA.at[pl.ds(0, BLK)]

        z16 = jnp.zeros((L,), jnp.float32)

        def zrow(i, carry):
            for j in range(D // L):
                rowA[i, pl.ds(j * L, L)] = z16
            return carry

        lax.fori_loop(0, BLK, zrow, 0)
        for t in range(NT):
            b = t * NS + s

            @pl.when(b < NBLK)
            def _():
                pltpu.sync_copy(stage_v, agg_sh.at[pl.ds(b * BLK, BLK)])

        plsc.subcore_barrier()

        def build(src_v, et_v, drw_v, gix_v, dst_v):
            for i in range(CH // L):
                sl = pl.ds(i * L, L)
                gix_v[sl] = et_v[sl] * N + src_v[sl]
                dst_v[sl] = drw_v[sl]

        def fetch_idx(cid, src_v, et_v, drw_v):
            base = cid * CH
            pltpu.async_copy(ei_hbm.at[pl.ds(base, CH)], src_v, isem)
            pltpu.async_copy(et_hbm.at[pl.ds(base, CH)], et_v, isem)
            pltpu.async_copy(ei_hbm.at[pl.ds(E + base, CH)], drw_v, isem)

        def drain_idx(src_v, et_v, drw_v):
            pltpu.make_async_copy(ei_hbm.at[pl.ds(0, CH)], src_v, isem).wait()
            pltpu.make_async_copy(et_hbm.at[pl.ds(0, CH)], et_v, isem).wait()
            pltpu.make_async_copy(ei_hbm.at[pl.ds(0, CH)], drw_v, isem).wait()

        fetch_idx(wid, srcA, etA, drwA)
        drain_idx(srcA, etA, drwA)
        build(srcA, etA, drwA, gixA, dstA)
        pltpu.async_copy(xw_hbm.at[gixA], rowA, gsem)
        fetch_idx(wid + NW, srcB, etB, drwB)

        def half(j, cur):
            src_c, et_c, drw_c, gix_c, dst_c, row_c = (
                (srcA, etA, drwA, gixA, dstA, rowA) if cur == 0
                else (srcB, etB, drwB, gixB, dstB, rowB))
            src_n, et_n, drw_n, gix_n, dst_n, row_n = (
                (srcB, etB, drwB, gixB, dstB, rowB) if cur == 0
                else (srcA, etA, drwA, gixA, dstA, rowA))
            cid0 = wid + j * NW
            cid1 = wid + (j + 1) * NW
            cid2 = wid + (j + 2) * NW

            @pl.when((j >= 1) & (wid + (j - 1) * NW < NCHUNK))
            def _():
                pltpu.make_async_copy(row_n, agg_sh.at[dst_n], ssem).wait()

            @pl.when(cid1 < NCHUNK)
            def _():
                drain_idx(src_n, et_n, drw_n)
                build(src_n, et_n, drw_n, gix_n, dst_n)

            @pl.when(cid0 < NCHUNK)
            def _():
                pltpu.make_async_copy(xw_hbm.at[gix_c], row_c, gsem).wait()
                pltpu.async_copy(row_c, agg_sh.at[dst_c], ssem, add=True)

            @pl.when(cid1 < NCHUNK)
            def _():
                pltpu.async_copy(xw_hbm.at[gix_n], row_n, gsem)

            @pl.when(cid2 < NCHUNK)
            def _():
                fetch_idx(cid2, src_c, et_c, drw_c)

        def body(jj, carry):
            half(2 * jj, 0)
            half(2 * jj + 1, 1)
            return carry

        lax.fori_loop(0, NJP // 2, body, 0)
        plsc.subcore_barrier()

        for t in range(NT):
            b = t * NS + s

            @pl.when(b < NBLK)
            def _():
                r0 = b * BLK
                pltpu.sync_copy(agg_sh.at[pl.ds(r0, BLK)],
                                out_hbm.at[c, pl.ds(r0, BLK)])

    return k(xw, ei, et)


def _mlp(x, partials, lw, rb, w1xt, w1mt, b1, w2xt, w2mt, b2):
    BN = 1000

    def body(x_ref, p_ref, lw_ref, rb_ref, w1x_ref, w1m_ref, b1_ref,
             w2x_ref, w2m_ref, b2_ref, o_ref):
        xb = x_ref[...]
        msg = (p_ref[0] + p_ref[1]
               + jnp.dot(xb, lw_ref[...], preferred_element_type=jnp.float32)
               + rb_ref[...])
        h = (jnp.dot(xb, w1x_ref[...], preferred_element_type=jnp.float32)
             + jnp.dot(msg, w1m_ref[...], preferred_element_type=jnp.float32)
             + b1_ref[...])
        mid = jnp.tanh(h)
        o_ref[...] = (jnp.dot(xb, w2x_ref[...], preferred_element_type=jnp.float32)
                      + jnp.dot(mid, w2m_ref[...], preferred_element_type=jnp.float32)
                      + b2_ref[...])

    return pl.pallas_call(
        body,
        grid=(N // BN,),
        in_specs=[
            pl.BlockSpec((BN, D), lambda i: (i, 0)),
            pl.BlockSpec((NC, BN, D), lambda i: (0, i, 0)),
            pl.BlockSpec((D, D), lambda i: (0, 0)),
            pl.BlockSpec((1, D), lambda i: (0, 0)),
            pl.BlockSpec((D, 2 * D), lambda i: (0, 0)),
            pl.BlockSpec((D, 2 * D), lambda i: (0, 0)),
            pl.BlockSpec((1, 2 * D), lambda i: (0, 0)),
            pl.BlockSpec((D, D), lambda i: (0, 0)),
            pl.BlockSpec((2 * D, D), lambda i: (0, 0)),
            pl.BlockSpec((1, D), lambda i: (0, 0)),
        ],
        out_specs=pl.BlockSpec((BN, D), lambda i: (i, 0)),
        out_shape=jax.ShapeDtypeStruct((N, D), jnp.float32),
    )(x, partials, lw, rb.reshape(1, D), w1xt, w1mt, b1.reshape(1, 2 * D),
      w2xt, w2mt, b2.reshape(1, D))


def kernel(x, edge_index, edges_type, is_block, rel_weight, loop_weight,
           rel_bias, W1, b1, W2, b2):
    del is_block
    ei = edge_index.astype(jnp.int32).reshape(2 * E)
    et = edges_type.astype(jnp.int32)
    xw = _relmm(x, rel_weight).reshape(R * N, D)
    partials = _sc_agg(xw, ei, et)
    w1xt = W1[:, :D].T
    w1mt = W1[:, D:].T
    w2xt = W2[:, :D].T
    w2mt = W2[:, D:].T
    return _mlp(x, partials, loop_weight, rel_bias, w1xt, w1mt, b1,
                w2xt, w2mt, b2)
